# baseline (speedup 1.0000x reference)
import jax
import jax.numpy as jnp
from jax import lax
from jax.experimental import pallas as pl
from jax.experimental.pallas import tpu as pltpu
from jax.experimental.pallas import tpu_sc as plsc

ALPHA = 0.0813
BETA = 0.947

NC, NS, L = 2, 16, 16
NW = NC * NS

N = 10000
E = 320000
D = 128
NP = 10240
RPT = NP // NS
B1 = 80
C1 = (E // NW) // B1
C2 = (E // NS) // B1
CP = (2 * E // NW) // B1


def _zero_fill(ref, nwords):
    def body(i, c):
        ref[pl.ds(i * L, L)] = jnp.zeros((L,), jnp.float32)
        return c
    lax.fori_loop(0, nwords // L, body, 0)


def _deg_body(dst_hbm, deg_hbm, dst_v, ones_v, zero_v, acc_s, bounce_v):
    cid = lax.axis_index("c")
    sid = lax.axis_index("s")
    wid = cid * NS + sid

    _zero_fill(zero_v, RPT)
    pltpu.sync_copy(zero_v, acc_s.at[pl.ds(sid * RPT, RPT)])
    plsc.subcore_barrier()

    pltpu.sync_copy(dst_hbm.at[wid], dst_v)
    for i in range(0, B1, L):
        ones_v[pl.ds(i, L)] = jnp.ones((L,), jnp.float32)

    def chunk(j, carry):
        pltpu.sync_copy(ones_v, acc_s.at[dst_v.at[j]], add=True)
        return carry

    lax.fori_loop(0, C1, chunk, 0)
    plsc.subcore_barrier()

    pltpu.sync_copy(acc_s.at[pl.ds(sid * RPT, RPT)], bounce_v)
    pltpu.sync_copy(bounce_v, deg_hbm.at[pl.ds(cid * NP + sid * RPT, RPT)])


def _deg_partials(dst_r):
    mesh = plsc.VectorSubcoreMesh(core_axis_name="c", subcore_axis_name="s")
    return pl.kernel(
        _deg_body,
        out_type=jax.ShapeDtypeStruct((NC * NP,), jnp.float32),
        mesh=mesh,
        compiler_params=pltpu.CompilerParams(needs_layout_passes=False),
        scratch_types=[
            pltpu.VMEM((C1, B1), jnp.int32),
            pltpu.VMEM((B1,), jnp.float32),
            pltpu.VMEM((RPT,), jnp.float32),
            pltpu.VMEM_SHARED((NP,), jnp.float32),
            pltpu.VMEM((RPT,), jnp.float32),
        ],
    )(dst_r)


def _spmm_body(xs_hbm, src_hbm, dst_hbm, zeros_hbm, out_hbm,
               src_v, dst_v, r2, acc_s, sem):
    cid = lax.axis_index("c")
    sid = lax.axis_index("s")
    wid = cid * NS + sid

    pltpu.sync_copy(zeros_hbm, r2)
    for t in range(RPT // B1):
        pltpu.sync_copy(r2, acc_s.at[pl.ds(sid * RPT + t * B1, B1)])
    plsc.subcore_barrier()

    pltpu.sync_copy(src_hbm.at[wid], src_v)
    pltpu.sync_copy(dst_hbm.at[wid], dst_v)

    def chunk(j, carry):
        pltpu.async_copy(xs_hbm.at[src_v.at[j]], r2, sem).wait()
        pltpu.sync_copy(r2, acc_s.at[dst_v.at[j]], add=True)
        return carry

    lax.fori_loop(0, C1, chunk, 0)
    plsc.subcore_barrier()

    for t in range(RPT // B1):
        base = sid * RPT + t * B1
        pltpu.sync_copy(acc_s.at[pl.ds(base, B1)], r2)
        pltpu.sync_copy(r2, out_hbm.at[pl.ds(cid * NP + base, B1)])


def _spmm_partials(xs, src_r, dst_r, zeros_bd):
    mesh = plsc.VectorSubcoreMesh(core_axis_name="c", subcore_axis_name="s")
    return pl.kernel(
        _spmm_body,
        out_type=jax.ShapeDtypeStruct((NC * NP, D), jnp.float32),
        mesh=mesh,
        compiler_params=pltpu.CompilerParams(needs_layout_passes=False),
        scratch_types=[
            pltpu.VMEM((C1, B1), jnp.int32),
            pltpu.VMEM((C1, B1), jnp.int32),
            pltpu.VMEM((B1, D), jnp.float32),
            pltpu.VMEM_SHARED((NP, D), jnp.float32),
            pltpu.SemaphoreType.DMA,
        ],
    )(xs, src_r, dst_r, zeros_bd)


def _l2_body(gsx_hbm, gsy_hbm, src2_hbm, dst2_hbm, t2x_hbm, t2y_hbm,
             dinv_hbm, pa_hbm, pb_hbm,
             embx_hbm, emby_hbm, ss_hbm,
             src2_v, dst2_v, rx_v, ry_v, zero_v,
             accx_s, accy_s, embx_s, emby_s,
             ax_v, ay_v, tx_v, ty_v, dv_v,
             exf_v, eyf_v, pj_v, qj_v, ss_v, semx, semy):
    cid = lax.axis_index("c")
    sid = lax.axis_index("s")
    wid = cid * NS + sid

    _zero_fill(zero_v, RPT)
    pltpu.sync_copy(zero_v, accx_s.at[pl.ds(sid * RPT, RPT)])
    pltpu.sync_copy(zero_v, accy_s.at[pl.ds(sid * RPT, RPT)])
    plsc.subcore_barrier()

    pltpu.sync_copy(src2_hbm.at[sid], src2_v)
    pltpu.sync_copy(dst2_hbm.at[sid], dst2_v)

    def chunk(j, carry):
        cx = pltpu.async_copy(gsx_hbm.at[src2_v.at[j]], rx_v, semx)
        cy = pltpu.async_copy(gsy_hbm.at[src2_v.at[j]], ry_v, semy)
        cx.wait()
        cy.wait()
        pltpu.sync_copy(rx_v, accx_s.at[dst2_v.at[j]], add=True)
        pltpu.sync_copy(ry_v, accy_s.at[dst2_v.at[j]], add=True)
        return carry

    lax.fori_loop(0, C2, chunk, 0)
    plsc.subcore_barrier()

    rbase = sid * RPT
    pltpu.sync_copy(accx_s.at[pl.ds(rbase, RPT)], ax_v)
    pltpu.sync_copy(accy_s.at[pl.ds(rbase, RPT)], ay_v)
    pltpu.sync_copy(t2x_hbm.at[pl.ds(rbase, RPT)], tx_v)
    pltpu.sync_copy(t2y_hbm.at[pl.ds(rbase, RPT)], ty_v)
    pltpu.sync_copy(dinv_hbm.at[pl.ds(rbase, RPT)], dv_v)

    def emb_row(k, carry):
        s = pl.ds(k * L, L)
        dv = dv_v[s]
        ax_v[s] = dv * ax_v[s] + tx_v[s]
        ay_v[s] = dv * ay_v[s] + ty_v[s]
        return carry

    lax.fori_loop(0, RPT // L, emb_row, 0)
    pltpu.sync_copy(ax_v, embx_s.at[pl.ds(rbase, RPT)])
    pltpu.sync_copy(ay_v, emby_s.at[pl.ds(rbase, RPT)])

    @pl.when(cid == 0)
    def _():
        pltpu.sync_copy(ax_v, embx_hbm.at[pl.ds(rbase, RPT)])
        pltpu.sync_copy(ay_v, emby_hbm.at[pl.ds(rbase, RPT)])

    plsc.subcore_barrier()

    pltpu.sync_copy(embx_s, exf_v)
    pltpu.sync_copy(emby_s, eyf_v)
    pbase = wid * (CP * B1)

    def pchunk(j, carry):
        pltpu.sync_copy(pa_hbm.at[pl.ds(pbase + j * B1, B1)], pj_v)
        pltpu.sync_copy(pb_hbm.at[pl.ds(pbase + j * B1, B1)], qj_v)
        for m in range(B1 // L):
            s = pl.ds(m * L, L)
            a = pj_v[s]
            b = qj_v[s]
            dx = plsc.load_gather(exf_v, [a]) - plsc.load_gather(exf_v, [b])
            dy = plsc.load_gather(eyf_v, [a]) - plsc.load_gather(eyf_v, [b])
            ss_v[s] = dx * dx + dy * dy
        pltpu.sync_copy(ss_v, ss_hbm.at[pl.ds(pbase + j * B1, B1)])
        return carry

    lax.fori_loop(0, CP, pchunk, 0)


def _layer2_and_dist(gsx, gsy, src2_r, dst2_r, t2x, t2y, dinv, pa, pb):
    mesh = plsc.VectorSubcoreMesh(core_axis_name="c", subcore_axis_name="s")
    return pl.kernel(
        _l2_body,
        out_type=(
            jax.ShapeDtypeStruct((NP,), jnp.float32),
            jax.ShapeDtypeStruct((NP,), jnp.float32),
            jax.ShapeDtypeStruct((2 * E,), jnp.float32),
        ),
        mesh=mesh,
        compiler_params=pltpu.CompilerParams(needs_layout_passes=False),
        scratch_types=[
            pltpu.VMEM((C2, B1), jnp.int32),
            pltpu.VMEM((C2, B1), jnp.int32),
            pltpu.VMEM((B1,), jnp.float32),
            pltpu.VMEM((B1,), jnp.float32),
            pltpu.VMEM((RPT,), jnp.float32),
            pltpu.VMEM_SHARED((NP,), jnp.float32),
            pltpu.VMEM_SHARED((NP,), jnp.float32),
            pltpu.VMEM_SHARED((NP,), jnp.float32),
            pltpu.VMEM_SHARED((NP,), jnp.float32),
            pltpu.VMEM((RPT,), jnp.float32),
            pltpu.VMEM((RPT,), jnp.float32),
            pltpu.VMEM((RPT,), jnp.float32),
            pltpu.VMEM((RPT,), jnp.float32),
            pltpu.VMEM((RPT,), jnp.float32),
            pltpu.VMEM((NP,), jnp.float32),
            pltpu.VMEM((NP,), jnp.float32),
            pltpu.VMEM((B1,), jnp.int32),
            pltpu.VMEM((B1,), jnp.int32),
            pltpu.VMEM((B1,), jnp.float32),
            pltpu.SemaphoreType.DMA,
            pltpu.SemaphoreType.DMA,
        ],
    )(gsx, gsy, src2_r, dst2_r, t2x, t2y, dinv, pa, pb)


def _prep_body(degp_ref, x_ref, xs_ref, dinv_ref, dinv2_ref):
    deg = degp_ref[0, :] + degp_ref[1, :] + 1.0
    dinv = lax.rsqrt(deg)
    dinv_ref[...] = dinv
    dinv2_ref[...] = dinv * dinv
    xs_ref[...] = dinv[:, None] * x_ref[...]


def _prep(deg_p, x_pad):
    return pl.pallas_call(
        _prep_body,
        out_shape=(
            jax.ShapeDtypeStruct((NP, D), jnp.float32),
            jax.ShapeDtypeStruct((NP,), jnp.float32),
            jax.ShapeDtypeStruct((NP,), jnp.float32),
        ),
    )(deg_p.reshape(NC, NP), x_pad)


_RB = 1024


def _mlp_body(p_ref, x_ref, dinv_ref, dinv2_ref, w1_ref, b1_ref, w2_ref,
              b2_ref, gsx_ref, gsy_ref, t2x_ref, t2y_ref):
    dinv = dinv_ref[...]
    dinv2 = dinv2_ref[...]
    out1 = (dinv[:, None] * (p_ref[0, :, :] + p_ref[1, :, :])
            + dinv2[:, None] * x_ref[...])
    h = jnp.maximum(
        jnp.dot(out1, w1_ref[...], preferred_element_type=jnp.float32)
        + b1_ref[...], 0.0)
    g = jnp.dot(h, w2_ref[...], preferred_element_type=jnp.float32)
    gsx_ref[...] = dinv * g[:, 0]
    gsy_ref[...] = dinv * g[:, 1]
    t2x_ref[...] = dinv2 * g[:, 0] + b2_ref[0]
    t2y_ref[...] = dinv2 * g[:, 1] + b2_ref[1]


def _mlp(acc_p, x_pad, dinv, dinv2, W1, b1, W2, b2):
    grid = NP // _RB
    return pl.pallas_call(
        _mlp_body,
        grid=(grid,),
        in_specs=[
            pl.BlockSpec((NC, _RB, D), lambda i: (0, i, 0)),
            pl.BlockSpec((_RB, D), lambda i: (i, 0)),
            pl.BlockSpec((_RB,), lambda i: (i,)),
            pl.BlockSpec((_RB,), lambda i: (i,)),
            pl.BlockSpec((D, 256), lambda i: (0, 0)),
            pl.BlockSpec((256,), lambda i: (0,)),
            pl.BlockSpec((256, 2), lambda i: (0, 0)),
            pl.BlockSpec((2,), lambda i: (0,)),
        ],
        out_specs=[
            pl.BlockSpec((_RB,), lambda i: (i,)),
            pl.BlockSpec((_RB,), lambda i: (i,)),
            pl.BlockSpec((_RB,), lambda i: (i,)),
            pl.BlockSpec((_RB,), lambda i: (i,)),
        ],
        out_shape=[
            jax.ShapeDtypeStruct((NP,), jnp.float32),
            jax.ShapeDtypeStruct((NP,), jnp.float32),
            jax.ShapeDtypeStruct((NP,), jnp.float32),
            jax.ShapeDtypeStruct((NP,), jnp.float32),
        ],
    )(acc_p.reshape(NC, NP, D), x_pad, dinv, dinv2, W1, b1, W2, b2)


_QB = 128000


def _q_body(ss_ref, q_ref):
    ss = ss_ref[...]
    q_ref[...] = 1.0 / (1.0 + ALPHA * jnp.power(ss + 1e-12, BETA))


def _q_from_ss(ss):
    return pl.pallas_call(
        _q_body,
        grid=(2 * E // _QB,),
        in_specs=[pl.BlockSpec((_QB,), lambda i: (i,))],
        out_specs=pl.BlockSpec((_QB,), lambda i: (i,)),
        out_shape=jax.ShapeDtypeStruct((2 * E,), jnp.float32),
    )(ss)


def kernel(features, edge_index, row_neg, col_neg, W1, b1, W2, b2):
    src = edge_index[0]
    dst = edge_index[1]

    dst_r = dst.reshape(NW, C1, B1)
    src_r = src.reshape(NW, C1, B1)
    src2_r = src.reshape(NS, C2, B1)
    dst2_r = dst.reshape(NS, C2, B1)
    pa = jnp.concatenate([src, row_neg], axis=0)
    pb = jnp.concatenate([dst, col_neg], axis=0)

    x_pad = jnp.pad(features, ((0, NP - N), (0, 0)))

    deg_p = _deg_partials(dst_r)
    xs, dinv, dinv2 = _prep(deg_p, x_pad)

    acc_p = _spmm_partials(xs, src_r, dst_r, jnp.zeros((B1, D), jnp.float32))
    gsx, gsy, t2x, t2y = _mlp(acc_p, x_pad, dinv, dinv2, W1, b1, W2, b2)

    embx, emby, ss = _layer2_and_dist(
        gsx, gsy, src2_r, dst2_r, t2x, t2y, dinv, pa, pb)
    emb = jnp.stack([embx[:N], emby[:N]], axis=1)

    q = _q_from_ss(ss)
    return (emb, q)


# speedup vs baseline: 3.0110x; 3.0110x over previous
import jax
import jax.numpy as jnp
from jax import lax
from jax.experimental import pallas as pl
from jax.experimental.pallas import tpu as pltpu
from jax.experimental.pallas import tpu_sc as plsc

ALPHA = 0.0813
BETA = 0.947

NC, NS, L = 2, 16, 16
NW = NC * NS

N = 10000
E = 320000
D = 128
NP = 10240
RPT = NP // NS
B1 = 80
C1 = (E // NW) // B1
C2 = (E // NS) // B1
CP = (2 * E // NW) // B1
BS = 40
CS = (E // NW) // BS


def _zero_fill(ref, nwords):
    def body(i, c):
        ref[pl.ds(i * L, L)] = jnp.zeros((L,), jnp.float32)
        return c
    lax.fori_loop(0, nwords // L, body, 0)


def _deg_body(dst_hbm, deg_hbm, dst_v, ones_v, zero_v, acc_s, bounce_v):
    cid = lax.axis_index("c")
    sid = lax.axis_index("s")
    wid = cid * NS + sid

    _zero_fill(zero_v, RPT)
    pltpu.sync_copy(zero_v, acc_s.at[pl.ds(sid * RPT, RPT)])
    plsc.subcore_barrier()

    pltpu.sync_copy(dst_hbm.at[wid], dst_v)
    for i in range(0, B1, L):
        ones_v[pl.ds(i, L)] = jnp.ones((L,), jnp.float32)

    def chunk(j, carry):
        pltpu.sync_copy(ones_v, acc_s.at[dst_v.at[j]], add=True)
        return carry

    lax.fori_loop(0, C1, chunk, 0)
    plsc.subcore_barrier()

    pltpu.sync_copy(acc_s.at[pl.ds(sid * RPT, RPT)], bounce_v)
    pltpu.sync_copy(bounce_v, deg_hbm.at[pl.ds(cid * NP + sid * RPT, RPT)])


def _deg_partials(dst_r):
    mesh = plsc.VectorSubcoreMesh(core_axis_name="c", subcore_axis_name="s")
    return pl.kernel(
        _deg_body,
        out_type=jax.ShapeDtypeStruct((NC * NP,), jnp.float32),
        mesh=mesh,
        compiler_params=pltpu.CompilerParams(needs_layout_passes=False),
        scratch_types=[
            pltpu.VMEM((C1, B1), jnp.int32),
            pltpu.VMEM((B1,), jnp.float32),
            pltpu.VMEM((RPT,), jnp.float32),
            pltpu.VMEM_SHARED((NP,), jnp.float32),
            pltpu.VMEM((RPT,), jnp.float32),
        ],
    )(dst_r)


SEG = 25
SEGS = C1 // SEG


def _spmm_body(xs_hbm, srcf_hbm, dst_hbm, zeros_hbm, out_hbm,
               srcf_v, dst_v, r2, rb, acc_s, sem, semb):
    cid = lax.axis_index("c")
    sid = lax.axis_index("s")
    wid = cid * NS + sid

    pltpu.sync_copy(zeros_hbm, r2)
    for t in range(RPT // B1):
        pltpu.sync_copy(r2, acc_s.at[pl.ds(sid * RPT + t * B1, B1)])
    plsc.subcore_barrier()

    pltpu.sync_copy(srcf_hbm.at[wid], srcf_v)
    pltpu.sync_copy(dst_hbm.at[wid], dst_v)

    pltpu.async_copy(xs_hbm.at[srcf_v.at[pl.ds(0, B1)]], r2, sem)

    def chunk(j, carry):
        sbase = j * B1

        @pl.when(j % 2 == 0)
        def _():
            @pl.when(j + 1 < C1)
            def _():
                pltpu.async_copy(
                    xs_hbm.at[srcf_v.at[pl.ds(sbase + B1, B1)]], rb, semb)
            pltpu.make_async_copy(xs_hbm.at[pl.ds(0, B1)], r2, sem).wait()
            pltpu.sync_copy(r2, acc_s.at[dst_v.at[j]], add=True)

        @pl.when(j % 2 == 1)
        def _():
            @pl.when(j + 1 < C1)
            def _():
                pltpu.async_copy(
                    xs_hbm.at[srcf_v.at[pl.ds(sbase + B1, B1)]], r2, sem)
            pltpu.make_async_copy(xs_hbm.at[pl.ds(0, B1)], rb, semb).wait()
            pltpu.sync_copy(rb, acc_s.at[dst_v.at[j]], add=True)

        return carry

    lax.fori_loop(0, C1, chunk, 0)
    plsc.subcore_barrier()

    for t in range(RPT // B1):
        base = sid * RPT + t * B1
        pltpu.sync_copy(acc_s.at[pl.ds(base, B1)], r2)
        pltpu.sync_copy(r2, out_hbm.at[pl.ds(cid * NP + base, B1)])


def _spmm_partials(xs, src_f, dst_r, zeros_bd):
    mesh = plsc.VectorSubcoreMesh(core_axis_name="c", subcore_axis_name="s")
    return pl.kernel(
        _spmm_body,
        out_type=jax.ShapeDtypeStruct((NC * NP, D), jnp.float32),
        mesh=mesh,
        compiler_params=pltpu.CompilerParams(needs_layout_passes=False),
        scratch_types=[
            pltpu.VMEM((E // NW,), jnp.int32),
            pltpu.VMEM((C1, B1), jnp.int32),
            pltpu.VMEM((B1, D), jnp.float32),
            pltpu.VMEM((B1, D), jnp.float32),
            pltpu.VMEM_SHARED((NP, D), jnp.float32),
            pltpu.SemaphoreType.DMA,
            pltpu.SemaphoreType.DMA,
        ],
    )(xs, src_f, dst_r, zeros_bd)


EPW = E // NW
PPW = 2 * E // NW


def _spmm2_body(gsx_hbm, gsy_hbm, srcf_hbm, dst_hbm, px_hbm, py_hbm,
                srcf_v, dst_v, gsx_v, gsy_v, rx_v, ry_v, zero_v,
                accx_s, accy_s, semx, semy):
    cid = lax.axis_index("c")
    sid = lax.axis_index("s")
    wid = cid * NS + sid

    _zero_fill(zero_v, RPT)
    pltpu.sync_copy(zero_v, accx_s.at[pl.ds(sid * RPT, RPT)])
    pltpu.sync_copy(zero_v, accy_s.at[pl.ds(sid * RPT, RPT)])
    pltpu.sync_copy(gsx_hbm, gsx_v)
    pltpu.sync_copy(gsy_hbm, gsy_v)
    pltpu.sync_copy(srcf_hbm.at[wid], srcf_v)
    pltpu.sync_copy(dst_hbm.at[wid], dst_v)
    plsc.subcore_barrier()

    def chunk(j, carry):
        for m in range(B1 // L):
            s = pl.ds(m * L, L)
            si = srcf_v[pl.ds(j * B1 + m * L, L)]
            rx_v[s] = plsc.load_gather(gsx_v, [si])
            ry_v[s] = plsc.load_gather(gsy_v, [si])
        cx = pltpu.async_copy(rx_v, accx_s.at[dst_v.at[j]], semx, add=True)
        cy = pltpu.async_copy(ry_v, accy_s.at[dst_v.at[j]], semy, add=True)
        cx.wait()
        cy.wait()
        return carry

    lax.fori_loop(0, C1, chunk, 0)
    plsc.subcore_barrier()

    rbase = sid * RPT
    pltpu.sync_copy(accx_s.at[pl.ds(rbase, RPT)], zero_v)
    pltpu.sync_copy(zero_v, px_hbm.at[pl.ds(cid * NP + rbase, RPT)])
    pltpu.sync_copy(accy_s.at[pl.ds(rbase, RPT)], zero_v)
    pltpu.sync_copy(zero_v, py_hbm.at[pl.ds(cid * NP + rbase, RPT)])


def _spmm2_partials(gsx, gsy, src_f, dst_r):
    mesh = plsc.VectorSubcoreMesh(core_axis_name="c", subcore_axis_name="s")
    return pl.kernel(
        _spmm2_body,
        out_type=(
            jax.ShapeDtypeStruct((NC * NP,), jnp.float32),
            jax.ShapeDtypeStruct((NC * NP,), jnp.float32),
        ),
        mesh=mesh,
        compiler_params=pltpu.CompilerParams(needs_layout_passes=False),
        scratch_types=[
            pltpu.VMEM((EPW,), jnp.int32),
            pltpu.VMEM((C1, B1), jnp.int32),
            pltpu.VMEM((NP,), jnp.float32),
            pltpu.VMEM((NP,), jnp.float32),
            pltpu.VMEM((B1,), jnp.float32),
            pltpu.VMEM((B1,), jnp.float32),
            pltpu.VMEM((RPT,), jnp.float32),
            pltpu.VMEM_SHARED((NP,), jnp.float32),
            pltpu.VMEM_SHARED((NP,), jnp.float32),
            pltpu.SemaphoreType.DMA,
            pltpu.SemaphoreType.DMA,
        ],
    )(gsx, gsy, src_f, dst_r)


def _embdist_body(px_hbm, py_hbm, t2x_hbm, t2y_hbm, dinv_hbm, pa_hbm, pb_hbm,
                  embx_hbm, emby_hbm, ss_hbm,
                  ax_v, ay_v, bx_v, by_v, dv_v,
                  embx_s, emby_s, exf_v, eyf_v, paf_v, pbf_v, ssw_v):
    cid = lax.axis_index("c")
    sid = lax.axis_index("s")
    wid = cid * NS + sid

    rbase = sid * RPT
    pltpu.sync_copy(px_hbm.at[pl.ds(rbase, RPT)], ax_v)
    pltpu.sync_copy(px_hbm.at[pl.ds(NP + rbase, RPT)], bx_v)
    pltpu.sync_copy(py_hbm.at[pl.ds(rbase, RPT)], ay_v)
    pltpu.sync_copy(py_hbm.at[pl.ds(NP + rbase, RPT)], by_v)
    pltpu.sync_copy(dinv_hbm.at[pl.ds(rbase, RPT)], dv_v)

    def emb_rowx(k, carry):
        s = pl.ds(k * L, L)
        ax_v[s] = dv_v[s] * (ax_v[s] + bx_v[s])
        ay_v[s] = dv_v[s] * (ay_v[s] + by_v[s])
        return carry

    lax.fori_loop(0, RPT // L, emb_rowx, 0)
    pltpu.sync_copy(t2x_hbm.at[pl.ds(rbase, RPT)], bx_v)
    pltpu.sync_copy(t2y_hbm.at[pl.ds(rbase, RPT)], by_v)

    def emb_rowy(k, carry):
        s = pl.ds(k * L, L)
        ax_v[s] = ax_v[s] + bx_v[s]
        ay_v[s] = ay_v[s] + by_v[s]
        return carry

    lax.fori_loop(0, RPT // L, emb_rowy, 0)
    pltpu.sync_copy(ax_v, embx_s.at[pl.ds(rbase, RPT)])
    pltpu.sync_copy(ay_v, emby_s.at[pl.ds(rbase, RPT)])

    @pl.when(cid == 0)
    def _():
        pltpu.sync_copy(ax_v, embx_hbm.at[pl.ds(rbase, RPT)])
        pltpu.sync_copy(ay_v, emby_hbm.at[pl.ds(rbase, RPT)])

    plsc.subcore_barrier()

    pltpu.sync_copy(embx_s, exf_v)
    pltpu.sync_copy(emby_s, eyf_v)
    pbase = wid * PPW
    pltpu.sync_copy(pa_hbm.at[pl.ds(pbase, PPW)], paf_v)
    pltpu.sync_copy(pb_hbm.at[pl.ds(pbase, PPW)], pbf_v)

    def pvec(v, carry):
        s = pl.ds(v * L, L)
        a = paf_v[s]
        b = pbf_v[s]
        dx = plsc.load_gather(exf_v, [a]) - plsc.load_gather(exf_v, [b])
        dy = plsc.load_gather(eyf_v, [a]) - plsc.load_gather(eyf_v, [b])
        ssw_v[s] = dx * dx + dy * dy
        return carry

    lax.fori_loop(0, PPW // L, pvec, 0)
    pltpu.sync_copy(ssw_v, ss_hbm.at[pl.ds(pbase, PPW)])


def _embdist(px, py, t2x, t2y, dinv, pa, pb):
    mesh = plsc.VectorSubcoreMesh(core_axis_name="c", subcore_axis_name="s")
    return pl.kernel(
        _embdist_body,
        out_type=(
            jax.ShapeDtypeStruct((NP,), jnp.float32),
            jax.ShapeDtypeStruct((NP,), jnp.float32),
            jax.ShapeDtypeStruct((2 * E,), jnp.float32),
        ),
        mesh=mesh,
        compiler_params=pltpu.CompilerParams(needs_layout_passes=False),
        scratch_types=[
            pltpu.VMEM((RPT,), jnp.float32),
            pltpu.VMEM((RPT,), jnp.float32),
            pltpu.VMEM((RPT,), jnp.float32),
            pltpu.VMEM((RPT,), jnp.float32),
            pltpu.VMEM((RPT,), jnp.float32),
            pltpu.VMEM_SHARED((NP,), jnp.float32),
            pltpu.VMEM_SHARED((NP,), jnp.float32),
            pltpu.VMEM((NP,), jnp.float32),
            pltpu.VMEM((NP,), jnp.float32),
            pltpu.VMEM((PPW,), jnp.int32),
            pltpu.VMEM((PPW,), jnp.int32),
            pltpu.VMEM((PPW,), jnp.float32),
        ],
    )(px, py, t2x, t2y, dinv, pa, pb)


def _prep_body(degp_ref, x_ref, xs_ref, dinv_ref, dinv2_ref):
    deg = degp_ref[0, :] + degp_ref[1, :] + 1.0
    dinv = lax.rsqrt(deg)
    dinv_ref[...] = dinv
    dinv2_ref[...] = dinv * dinv
    xs_ref[...] = dinv[:, None] * x_ref[...]


def _prep(deg_p, x_pad):
    return pl.pallas_call(
        _prep_body,
        out_shape=(
            jax.ShapeDtypeStruct((NP, D), jnp.float32),
            jax.ShapeDtypeStruct((NP,), jnp.float32),
            jax.ShapeDtypeStruct((NP,), jnp.float32),
        ),
    )(deg_p.reshape(NC, NP), x_pad)


_RB = 1024


def _mlp_body(p_ref, x_ref, dinv_ref, dinv2_ref, w1_ref, b1_ref, w2_ref,
              b2_ref, gsx_ref, gsy_ref, t2x_ref, t2y_ref):
    dinv = dinv_ref[...]
    dinv2 = dinv2_ref[...]
    out1 = (dinv[:, None] * (p_ref[0, :, :] + p_ref[1, :, :])
            + dinv2[:, None] * x_ref[...])
    h = jnp.maximum(
        jnp.dot(out1, w1_ref[...], preferred_element_type=jnp.float32)
        + b1_ref[...], 0.0)
    g = jnp.dot(h, w2_ref[...], preferred_element_type=jnp.float32)
    gsx_ref[...] = dinv * g[:, 0]
    gsy_ref[...] = dinv * g[:, 1]
    t2x_ref[...] = dinv2 * g[:, 0] + b2_ref[0]
    t2y_ref[...] = dinv2 * g[:, 1] + b2_ref[1]


def _mlp(acc_p, x_pad, dinv, dinv2, W1, b1, W2, b2):
    grid = NP // _RB
    return pl.pallas_call(
        _mlp_body,
        grid=(grid,),
        in_specs=[
            pl.BlockSpec((NC, _RB, D), lambda i: (0, i, 0)),
            pl.BlockSpec((_RB, D), lambda i: (i, 0)),
            pl.BlockSpec((_RB,), lambda i: (i,)),
            pl.BlockSpec((_RB,), lambda i: (i,)),
            pl.BlockSpec((D, 256), lambda i: (0, 0)),
            pl.BlockSpec((256,), lambda i: (0,)),
            pl.BlockSpec((256, 2), lambda i: (0, 0)),
            pl.BlockSpec((2,), lambda i: (0,)),
        ],
        out_specs=[
            pl.BlockSpec((_RB,), lambda i: (i,)),
            pl.BlockSpec((_RB,), lambda i: (i,)),
            pl.BlockSpec((_RB,), lambda i: (i,)),
            pl.BlockSpec((_RB,), lambda i: (i,)),
        ],
        out_shape=[
            jax.ShapeDtypeStruct((NP,), jnp.float32),
            jax.ShapeDtypeStruct((NP,), jnp.float32),
            jax.ShapeDtypeStruct((NP,), jnp.float32),
            jax.ShapeDtypeStruct((NP,), jnp.float32),
        ],
    )(acc_p.reshape(NC, NP, D), x_pad, dinv, dinv2, W1, b1, W2, b2)


_QB = 128000


def _q_body(ss_ref, q_ref):
    ss = ss_ref[...]
    q_ref[...] = 1.0 / (1.0 + ALPHA * jnp.power(ss + 1e-12, BETA))


def _q_from_ss(ss):
    return pl.pallas_call(
        _q_body,
        grid=(2 * E // _QB,),
        in_specs=[pl.BlockSpec((_QB,), lambda i: (i,))],
        out_specs=pl.BlockSpec((_QB,), lambda i: (i,)),
        out_shape=jax.ShapeDtypeStruct((2 * E,), jnp.float32),
    )(ss)


def kernel(features, edge_index, row_neg, col_neg, W1, b1, W2, b2):
    src = edge_index[0]
    dst = edge_index[1]

    dst_r = dst.reshape(NW, C1, B1)
    src_f = src.reshape(NW, EPW)
    pa = jnp.concatenate([src, row_neg], axis=0)
    pb = jnp.concatenate([dst, col_neg], axis=0)

    x_pad = jnp.pad(features, ((0, NP - N), (0, 0)))

    deg_p = _deg_partials(dst_r)
    xs, dinv, dinv2 = _prep(deg_p, x_pad)

    acc_p = _spmm_partials(xs, src_f, dst_r, jnp.zeros((B1, D), jnp.float32))
    gsx, gsy, t2x, t2y = _mlp(acc_p, x_pad, dinv, dinv2, W1, b1, W2, b2)

    px, py = _spmm2_partials(gsx, gsy, src_f, dst_r)
    embx, emby, ss = _embdist(px, py, t2x, t2y, dinv, pa, pb)
    emb = jnp.stack([embx[:N], emby[:N]], axis=1)

    q = _q_from_ss(ss)
    return (emb, q)


# speedup vs baseline: 3.1783x; 1.0555x over previous
import jax
import jax.numpy as jnp
from jax import lax
from jax.experimental import pallas as pl
from jax.experimental.pallas import tpu as pltpu
from jax.experimental.pallas import tpu_sc as plsc

ALPHA = 0.0813
BETA = 0.947

NC, NS, L = 2, 16, 16
NW = NC * NS

N = 10000
E = 320000
D = 128
NP = 10240
RPT = NP // NS
B1 = 80
C1 = (E // NW) // B1
C2 = (E // NS) // B1
CP = (2 * E // NW) // B1
BS = 40
CS = (E // NW) // BS


def _zero_fill(ref, nwords):
    def body(i, c):
        ref[pl.ds(i * L, L)] = jnp.zeros((L,), jnp.float32)
        return c
    lax.fori_loop(0, nwords // L, body, 0)


def _deg_body(dst_hbm, deg_hbm, dst_v, ones_v, zero_v, acc_s, bounce_v, sem):
    cid = lax.axis_index("c")
    sid = lax.axis_index("s")
    wid = cid * NS + sid

    _zero_fill(zero_v, RPT)
    pltpu.sync_copy(zero_v, acc_s.at[pl.ds(sid * RPT, RPT)])
    plsc.subcore_barrier()

    pltpu.sync_copy(dst_hbm.at[wid], dst_v)
    for i in range(0, B1, L):
        ones_v[pl.ds(i, L)] = jnp.ones((L,), jnp.float32)

    def chunk(j, carry):
        pltpu.async_copy(ones_v, acc_s.at[dst_v.at[j]], sem, add=True)

        @pl.when(j >= 16)
        def _():
            pltpu.make_async_copy(ones_v, acc_s.at[dst_v.at[0]], sem).wait()
        return carry

    lax.fori_loop(0, C1, chunk, 0)

    def drain(j, carry):
        pltpu.make_async_copy(ones_v, acc_s.at[dst_v.at[0]], sem).wait()
        return carry

    lax.fori_loop(0, 16, drain, 0)
    plsc.subcore_barrier()

    pltpu.sync_copy(acc_s.at[pl.ds(sid * RPT, RPT)], bounce_v)
    pltpu.sync_copy(bounce_v, deg_hbm.at[pl.ds(cid * NP + sid * RPT, RPT)])


def _deg_partials(dst_r):
    mesh = plsc.VectorSubcoreMesh(core_axis_name="c", subcore_axis_name="s")
    return pl.kernel(
        _deg_body,
        out_type=jax.ShapeDtypeStruct((NC * NP,), jnp.float32),
        mesh=mesh,
        compiler_params=pltpu.CompilerParams(needs_layout_passes=False),
        scratch_types=[
            pltpu.VMEM((C1, B1), jnp.int32),
            pltpu.VMEM((B1,), jnp.float32),
            pltpu.VMEM((RPT,), jnp.float32),
            pltpu.VMEM_SHARED((NP,), jnp.float32),
            pltpu.VMEM((RPT,), jnp.float32),
            pltpu.SemaphoreType.DMA,
        ],
    )(dst_r)


SEG = 25
SEGS = C1 // SEG


def _spmm_body(xs_hbm, srcf_hbm, dst_hbm, zeros_hbm, out_hbm,
               srcf_v, dst_v, r2, rb, acc_s, sem, semb):
    cid = lax.axis_index("c")
    sid = lax.axis_index("s")
    wid = cid * NS + sid

    pltpu.sync_copy(zeros_hbm, r2)
    for t in range(RPT // B1):
        pltpu.sync_copy(r2, acc_s.at[pl.ds(sid * RPT + t * B1, B1)])
    plsc.subcore_barrier()

    pltpu.sync_copy(srcf_hbm.at[wid], srcf_v)
    pltpu.sync_copy(dst_hbm.at[wid], dst_v)

    pltpu.async_copy(xs_hbm.at[srcf_v.at[pl.ds(0, B1)]], r2, sem)

    def chunk(j, carry):
        sbase = j * B1

        @pl.when(j % 2 == 0)
        def _():
            @pl.when(j + 1 < C1)
            def _():
                pltpu.async_copy(
                    xs_hbm.at[srcf_v.at[pl.ds(sbase + B1, B1)]], rb, semb)
            pltpu.make_async_copy(xs_hbm.at[pl.ds(0, B1)], r2, sem).wait()
            pltpu.sync_copy(r2, acc_s.at[dst_v.at[j]], add=True)

        @pl.when(j % 2 == 1)
        def _():
            @pl.when(j + 1 < C1)
            def _():
                pltpu.async_copy(
                    xs_hbm.at[srcf_v.at[pl.ds(sbase + B1, B1)]], r2, sem)
            pltpu.make_async_copy(xs_hbm.at[pl.ds(0, B1)], rb, semb).wait()
            pltpu.sync_copy(rb, acc_s.at[dst_v.at[j]], add=True)

        return carry

    lax.fori_loop(0, C1, chunk, 0)
    plsc.subcore_barrier()

    for t in range(RPT // B1):
        base = sid * RPT + t * B1
        pltpu.sync_copy(acc_s.at[pl.ds(base, B1)], r2)
        pltpu.sync_copy(r2, out_hbm.at[pl.ds(cid * NP + base, B1)])


def _spmm_partials(xs, src_f, dst_r, zeros_bd):
    mesh = plsc.VectorSubcoreMesh(core_axis_name="c", subcore_axis_name="s")
    return pl.kernel(
        _spmm_body,
        out_type=jax.ShapeDtypeStruct((NC * NP, D), jnp.float32),
        mesh=mesh,
        compiler_params=pltpu.CompilerParams(needs_layout_passes=False),
        scratch_types=[
            pltpu.VMEM((E // NW,), jnp.int32),
            pltpu.VMEM((C1, B1), jnp.int32),
            pltpu.VMEM((B1, D), jnp.float32),
            pltpu.VMEM((B1, D), jnp.float32),
            pltpu.VMEM_SHARED((NP, D), jnp.float32),
            pltpu.SemaphoreType.DMA,
            pltpu.SemaphoreType.DMA,
        ],
    )(xs, src_f, dst_r, zeros_bd)


EPW = E // NW
PPW = 2 * E // NW


def _spmm2_body(gsx_hbm, gsy_hbm, srcf_hbm, dst_hbm, px_hbm, py_hbm,
                srcf_v, dst_v, gsx_v, gsy_v, rx_v, ry_v, rx2_v, ry2_v,
                zero_v, accx_s, accy_s, semx, semy, semx2, semy2):
    cid = lax.axis_index("c")
    sid = lax.axis_index("s")
    wid = cid * NS + sid

    _zero_fill(zero_v, RPT)
    pltpu.sync_copy(zero_v, accx_s.at[pl.ds(sid * RPT, RPT)])
    pltpu.sync_copy(zero_v, accy_s.at[pl.ds(sid * RPT, RPT)])
    pltpu.sync_copy(gsx_hbm, gsx_v)
    pltpu.sync_copy(gsy_hbm, gsy_v)
    pltpu.sync_copy(srcf_hbm.at[wid], srcf_v)
    pltpu.sync_copy(dst_hbm.at[wid], dst_v)
    plsc.subcore_barrier()

    def build_fire(j, rx, ry, sx, sy):
        for m in range(B1 // L):
            s = pl.ds(m * L, L)
            si = srcf_v[pl.ds(j * B1 + m * L, L)]
            rx[s] = plsc.load_gather(gsx_v, [si])
            ry[s] = plsc.load_gather(gsy_v, [si])
        pltpu.async_copy(rx, accx_s.at[dst_v.at[j]], sx, add=True)
        pltpu.async_copy(ry, accy_s.at[dst_v.at[j]], sy, add=True)

    def chunk(j, carry):
        @pl.when(j % 2 == 0)
        def _():
            @pl.when(j >= 2)
            def _():
                pltpu.make_async_copy(rx_v, accx_s.at[dst_v.at[0]],
                                      semx).wait()
                pltpu.make_async_copy(ry_v, accy_s.at[dst_v.at[0]],
                                      semy).wait()
            build_fire(j, rx_v, ry_v, semx, semy)

        @pl.when(j % 2 == 1)
        def _():
            @pl.when(j >= 2)
            def _():
                pltpu.make_async_copy(rx2_v, accx_s.at[dst_v.at[0]],
                                      semx2).wait()
                pltpu.make_async_copy(ry2_v, accy_s.at[dst_v.at[0]],
                                      semy2).wait()
            build_fire(j, rx2_v, ry2_v, semx2, semy2)

        return carry

    lax.fori_loop(0, C1, chunk, 0)
    pltpu.make_async_copy(rx_v, accx_s.at[dst_v.at[0]], semx).wait()
    pltpu.make_async_copy(ry_v, accy_s.at[dst_v.at[0]], semy).wait()
    pltpu.make_async_copy(rx2_v, accx_s.at[dst_v.at[0]], semx2).wait()
    pltpu.make_async_copy(ry2_v, accy_s.at[dst_v.at[0]], semy2).wait()
    plsc.subcore_barrier()

    rbase = sid * RPT
    pltpu.sync_copy(accx_s.at[pl.ds(rbase, RPT)], zero_v)
    pltpu.sync_copy(zero_v, px_hbm.at[pl.ds(cid * NP + rbase, RPT)])
    pltpu.sync_copy(accy_s.at[pl.ds(rbase, RPT)], zero_v)
    pltpu.sync_copy(zero_v, py_hbm.at[pl.ds(cid * NP + rbase, RPT)])


def _spmm2_partials(gsx, gsy, src_f, dst_r):
    mesh = plsc.VectorSubcoreMesh(core_axis_name="c", subcore_axis_name="s")
    return pl.kernel(
        _spmm2_body,
        out_type=(
            jax.ShapeDtypeStruct((NC * NP,), jnp.float32),
            jax.ShapeDtypeStruct((NC * NP,), jnp.float32),
        ),
        mesh=mesh,
        compiler_params=pltpu.CompilerParams(needs_layout_passes=False),
        scratch_types=[
            pltpu.VMEM((EPW,), jnp.int32),
            pltpu.VMEM((C1, B1), jnp.int32),
            pltpu.VMEM((NP,), jnp.float32),
            pltpu.VMEM((NP,), jnp.float32),
            pltpu.VMEM((B1,), jnp.float32),
            pltpu.VMEM((B1,), jnp.float32),
            pltpu.VMEM((B1,), jnp.float32),
            pltpu.VMEM((B1,), jnp.float32),
            pltpu.VMEM((RPT,), jnp.float32),
            pltpu.VMEM_SHARED((NP,), jnp.float32),
            pltpu.VMEM_SHARED((NP,), jnp.float32),
            pltpu.SemaphoreType.DMA,
            pltpu.SemaphoreType.DMA,
            pltpu.SemaphoreType.DMA,
            pltpu.SemaphoreType.DMA,
        ],
    )(gsx, gsy, src_f, dst_r)


def _embdist_body(px_hbm, py_hbm, t2x_hbm, t2y_hbm, dinv_hbm, pa_hbm, pb_hbm,
                  embx_hbm, emby_hbm, ss_hbm,
                  ax_v, ay_v, bx_v, by_v, dv_v,
                  embx_s, emby_s, exf_v, eyf_v, paf_v, pbf_v, ssw_v):
    cid = lax.axis_index("c")
    sid = lax.axis_index("s")
    wid = cid * NS + sid

    rbase = sid * RPT
    pltpu.sync_copy(px_hbm.at[pl.ds(rbase, RPT)], ax_v)
    pltpu.sync_copy(px_hbm.at[pl.ds(NP + rbase, RPT)], bx_v)
    pltpu.sync_copy(py_hbm.at[pl.ds(rbase, RPT)], ay_v)
    pltpu.sync_copy(py_hbm.at[pl.ds(NP + rbase, RPT)], by_v)
    pltpu.sync_copy(dinv_hbm.at[pl.ds(rbase, RPT)], dv_v)

    def emb_rowx(k, carry):
        s = pl.ds(k * L, L)
        ax_v[s] = dv_v[s] * (ax_v[s] + bx_v[s])
        ay_v[s] = dv_v[s] * (ay_v[s] + by_v[s])
        return carry

    lax.fori_loop(0, RPT // L, emb_rowx, 0)
    pltpu.sync_copy(t2x_hbm.at[pl.ds(rbase, RPT)], bx_v)
    pltpu.sync_copy(t2y_hbm.at[pl.ds(rbase, RPT)], by_v)

    def emb_rowy(k, carry):
        s = pl.ds(k * L, L)
        ax_v[s] = ax_v[s] + bx_v[s]
        ay_v[s] = ay_v[s] + by_v[s]
        return carry

    lax.fori_loop(0, RPT // L, emb_rowy, 0)
    pltpu.sync_copy(ax_v, embx_s.at[pl.ds(rbase, RPT)])
    pltpu.sync_copy(ay_v, emby_s.at[pl.ds(rbase, RPT)])

    @pl.when(cid == 0)
    def _():
        pltpu.sync_copy(ax_v, embx_hbm.at[pl.ds(rbase, RPT)])
        pltpu.sync_copy(ay_v, emby_hbm.at[pl.ds(rbase, RPT)])

    plsc.subcore_barrier()

    pltpu.sync_copy(embx_s, exf_v)
    pltpu.sync_copy(emby_s, eyf_v)
    pbase = wid * PPW
    pltpu.sync_copy(pa_hbm.at[pl.ds(pbase, PPW)], paf_v)
    pltpu.sync_copy(pb_hbm.at[pl.ds(pbase, PPW)], pbf_v)

    def pvec(v, carry):
        for r in range(5):
            s = pl.ds((v * 5 + r) * L, L)
            a = paf_v[s]
            b = pbf_v[s]
            dx = plsc.load_gather(exf_v, [a]) - plsc.load_gather(exf_v, [b])
            dy = plsc.load_gather(eyf_v, [a]) - plsc.load_gather(eyf_v, [b])
            ssw_v[s] = dx * dx + dy * dy
        return carry

    lax.fori_loop(0, PPW // L // 5, pvec, 0)
    pltpu.sync_copy(ssw_v, ss_hbm.at[pl.ds(pbase, PPW)])


def _embdist(px, py, t2x, t2y, dinv, pa, pb):
    mesh = plsc.VectorSubcoreMesh(core_axis_name="c", subcore_axis_name="s")
    return pl.kernel(
        _embdist_body,
        out_type=(
            jax.ShapeDtypeStruct((NP,), jnp.float32),
            jax.ShapeDtypeStruct((NP,), jnp.float32),
            jax.ShapeDtypeStruct((2 * E,), jnp.float32),
        ),
        mesh=mesh,
        compiler_params=pltpu.CompilerParams(needs_layout_passes=False),
        scratch_types=[
            pltpu.VMEM((RPT,), jnp.float32),
            pltpu.VMEM((RPT,), jnp.float32),
            pltpu.VMEM((RPT,), jnp.float32),
            pltpu.VMEM((RPT,), jnp.float32),
            pltpu.VMEM((RPT,), jnp.float32),
            pltpu.VMEM_SHARED((NP,), jnp.float32),
            pltpu.VMEM_SHARED((NP,), jnp.float32),
            pltpu.VMEM((NP,), jnp.float32),
            pltpu.VMEM((NP,), jnp.float32),
            pltpu.VMEM((PPW,), jnp.int32),
            pltpu.VMEM((PPW,), jnp.int32),
            pltpu.VMEM((PPW,), jnp.float32),
        ],
    )(px, py, t2x, t2y, dinv, pa, pb)


def _prep_body(degp_ref, x_ref, xs_ref, dinv_ref, dinv2_ref):
    deg = degp_ref[0, :] + degp_ref[1, :] + 1.0
    dinv = lax.rsqrt(deg)
    dinv_ref[...] = dinv
    dinv2_ref[...] = dinv * dinv
    xs_ref[...] = dinv[:, None] * x_ref[...]


def _prep(deg_p, x_pad):
    return pl.pallas_call(
        _prep_body,
        out_shape=(
            jax.ShapeDtypeStruct((NP, D), jnp.float32),
            jax.ShapeDtypeStruct((NP,), jnp.float32),
            jax.ShapeDtypeStruct((NP,), jnp.float32),
        ),
    )(deg_p.reshape(NC, NP), x_pad)


_RB = 1024


def _mlp_body(p_ref, x_ref, dinv_ref, dinv2_ref, w1_ref, b1_ref, w2_ref,
              b2_ref, gsx_ref, gsy_ref, t2x_ref, t2y_ref):
    dinv = dinv_ref[...]
    dinv2 = dinv2_ref[...]
    out1 = (dinv[:, None] * (p_ref[0, :, :] + p_ref[1, :, :])
            + dinv2[:, None] * x_ref[...])
    h = jnp.maximum(
        jnp.dot(out1, w1_ref[...], preferred_element_type=jnp.float32)
        + b1_ref[...], 0.0)
    g = jnp.dot(h, w2_ref[...], preferred_element_type=jnp.float32)
    gsx_ref[...] = dinv * g[:, 0]
    gsy_ref[...] = dinv * g[:, 1]
    t2x_ref[...] = dinv2 * g[:, 0] + b2_ref[0]
    t2y_ref[...] = dinv2 * g[:, 1] + b2_ref[1]


def _mlp(acc_p, x_pad, dinv, dinv2, W1, b1, W2, b2):
    grid = NP // _RB
    return pl.pallas_call(
        _mlp_body,
        grid=(grid,),
        in_specs=[
            pl.BlockSpec((NC, _RB, D), lambda i: (0, i, 0)),
            pl.BlockSpec((_RB, D), lambda i: (i, 0)),
            pl.BlockSpec((_RB,), lambda i: (i,)),
            pl.BlockSpec((_RB,), lambda i: (i,)),
            pl.BlockSpec((D, 256), lambda i: (0, 0)),
            pl.BlockSpec((256,), lambda i: (0,)),
            pl.BlockSpec((256, 2), lambda i: (0, 0)),
            pl.BlockSpec((2,), lambda i: (0,)),
        ],
        out_specs=[
            pl.BlockSpec((_RB,), lambda i: (i,)),
            pl.BlockSpec((_RB,), lambda i: (i,)),
            pl.BlockSpec((_RB,), lambda i: (i,)),
            pl.BlockSpec((_RB,), lambda i: (i,)),
        ],
        out_shape=[
            jax.ShapeDtypeStruct((NP,), jnp.float32),
            jax.ShapeDtypeStruct((NP,), jnp.float32),
            jax.ShapeDtypeStruct((NP,), jnp.float32),
            jax.ShapeDtypeStruct((NP,), jnp.float32),
        ],
    )(acc_p.reshape(NC, NP, D), x_pad, dinv, dinv2, W1, b1, W2, b2)


_QB = 128000


def _q_body(ss_ref, q_ref):
    ss = ss_ref[...]
    q_ref[...] = 1.0 / (1.0 + ALPHA * jnp.power(ss + 1e-12, BETA))


def _q_from_ss(ss):
    return pl.pallas_call(
        _q_body,
        grid=(2 * E // _QB,),
        in_specs=[pl.BlockSpec((_QB,), lambda i: (i,))],
        out_specs=pl.BlockSpec((_QB,), lambda i: (i,)),
        out_shape=jax.ShapeDtypeStruct((2 * E,), jnp.float32),
    )(ss)


def kernel(features, edge_index, row_neg, col_neg, W1, b1, W2, b2):
    src = edge_index[0]
    dst = edge_index[1]

    dst_r = dst.reshape(NW, C1, B1)
    src_f = src.reshape(NW, EPW)
    pa = jnp.concatenate([src, row_neg], axis=0)
    pb = jnp.concatenate([dst, col_neg], axis=0)

    x_pad = jnp.pad(features, ((0, NP - N), (0, 0)))

    deg_p = _deg_partials(dst_r)
    xs, dinv, dinv2 = _prep(deg_p, x_pad)

    acc_p = _spmm_partials(xs, src_f, dst_r, jnp.zeros((B1, D), jnp.float32))
    gsx, gsy, t2x, t2y = _mlp(acc_p, x_pad, dinv, dinv2, W1, b1, W2, b2)

    px, py = _spmm2_partials(gsx, gsy, src_f, dst_r)
    embx, emby, ss = _embdist(px, py, t2x, t2y, dinv, pa, pb)
    emb = jnp.stack([embx[:N], emby[:N]], axis=1)

    q = _q_from_ss(ss)
    return (emb, q)


# speedup vs baseline: 3.2404x; 1.0195x over previous
import jax
import jax.numpy as jnp
from jax import lax
from jax.experimental import pallas as pl
from jax.experimental.pallas import tpu as pltpu
from jax.experimental.pallas import tpu_sc as plsc

ALPHA = 0.0813
BETA = 0.947

NC, NS, L = 2, 16, 16
NW = NC * NS

N = 10000
E = 320000
D = 128
NP = 10240
RPT = NP // NS
B1 = 80
C1 = (E // NW) // B1
C2 = (E // NS) // B1
CP = (2 * E // NW) // B1
BS = 40
CS = (E // NW) // BS


def _zero_fill(ref, nwords):
    def body(i, c):
        ref[pl.ds(i * L, L)] = jnp.zeros((L,), jnp.float32)
        return c
    lax.fori_loop(0, nwords // L, body, 0)


def _deg_body(dst_hbm, deg_hbm, dst_v, ones_v, zero_v, acc_s, bounce_v, sem):
    cid = lax.axis_index("c")
    sid = lax.axis_index("s")
    wid = cid * NS + sid

    _zero_fill(zero_v, RPT)
    pltpu.sync_copy(zero_v, acc_s.at[pl.ds(sid * RPT, RPT)])
    plsc.subcore_barrier()

    pltpu.sync_copy(dst_hbm.at[wid], dst_v)
    for i in range(0, B1, L):
        ones_v[pl.ds(i, L)] = jnp.ones((L,), jnp.float32)

    def chunk(j, carry):
        pltpu.async_copy(ones_v, acc_s.at[dst_v.at[j]], sem, add=True)

        @pl.when(j >= 16)
        def _():
            pltpu.make_async_copy(ones_v, acc_s.at[dst_v.at[0]], sem).wait()
        return carry

    lax.fori_loop(0, C1, chunk, 0)

    def drain(j, carry):
        pltpu.make_async_copy(ones_v, acc_s.at[dst_v.at[0]], sem).wait()
        return carry

    lax.fori_loop(0, 16, drain, 0)
    plsc.subcore_barrier()

    pltpu.sync_copy(acc_s.at[pl.ds(sid * RPT, RPT)], bounce_v)
    pltpu.sync_copy(bounce_v, deg_hbm.at[pl.ds(cid * NP + sid * RPT, RPT)])


def _deg_partials(dst_r):
    mesh = plsc.VectorSubcoreMesh(core_axis_name="c", subcore_axis_name="s")
    return pl.kernel(
        _deg_body,
        out_type=jax.ShapeDtypeStruct((NC * NP,), jnp.float32),
        mesh=mesh,
        compiler_params=pltpu.CompilerParams(needs_layout_passes=False),
        scratch_types=[
            pltpu.VMEM((C1, B1), jnp.int32),
            pltpu.VMEM((B1,), jnp.float32),
            pltpu.VMEM((RPT,), jnp.float32),
            pltpu.VMEM_SHARED((NP,), jnp.float32),
            pltpu.VMEM((RPT,), jnp.float32),
            pltpu.SemaphoreType.DMA,
        ],
    )(dst_r)


SEG = 25
SEGS = C1 // SEG


def _spmm_body(xs_hbm, srcf_hbm, dst_hbm, zeros_hbm, out_hbm,
               srcf_v, dst_v, r2, rb, acc_s, sem, semb):
    cid = lax.axis_index("c")
    sid = lax.axis_index("s")
    wid = cid * NS + sid

    pltpu.sync_copy(zeros_hbm, r2)
    for t in range(RPT // B1):
        pltpu.sync_copy(r2, acc_s.at[pl.ds(sid * RPT + t * B1, B1)])
    plsc.subcore_barrier()

    pltpu.sync_copy(srcf_hbm.at[wid], srcf_v)
    pltpu.sync_copy(dst_hbm.at[wid], dst_v)

    pltpu.async_copy(xs_hbm.at[srcf_v.at[pl.ds(0, B1)]], r2, sem)

    def chunk(j, carry):
        sbase = j * B1

        @pl.when(j % 2 == 0)
        def _():
            @pl.when(j + 1 < C1)
            def _():
                pltpu.async_copy(
                    xs_hbm.at[srcf_v.at[pl.ds(sbase + B1, B1)]], rb, semb)
            pltpu.make_async_copy(xs_hbm.at[pl.ds(0, B1)], r2, sem).wait()
            pltpu.sync_copy(r2, acc_s.at[dst_v.at[j]], add=True)

        @pl.when(j % 2 == 1)
        def _():
            @pl.when(j + 1 < C1)
            def _():
                pltpu.async_copy(
                    xs_hbm.at[srcf_v.at[pl.ds(sbase + B1, B1)]], r2, sem)
            pltpu.make_async_copy(xs_hbm.at[pl.ds(0, B1)], rb, semb).wait()
            pltpu.sync_copy(rb, acc_s.at[dst_v.at[j]], add=True)

        return carry

    lax.fori_loop(0, C1, chunk, 0)
    plsc.subcore_barrier()

    for t in range(RPT // B1):
        base = sid * RPT + t * B1
        pltpu.sync_copy(acc_s.at[pl.ds(base, B1)], r2)
        pltpu.sync_copy(r2, out_hbm.at[pl.ds(cid * NP + base, B1)])


def _spmm_partials(xs, src_f, dst_r, zeros_bd):
    mesh = plsc.VectorSubcoreMesh(core_axis_name="c", subcore_axis_name="s")
    return pl.kernel(
        _spmm_body,
        out_type=jax.ShapeDtypeStruct((NC * NP, D), jnp.float32),
        mesh=mesh,
        compiler_params=pltpu.CompilerParams(needs_layout_passes=False),
        scratch_types=[
            pltpu.VMEM((E // NW,), jnp.int32),
            pltpu.VMEM((C1, B1), jnp.int32),
            pltpu.VMEM((B1, D), jnp.float32),
            pltpu.VMEM((B1, D), jnp.float32),
            pltpu.VMEM_SHARED((NP, D), jnp.float32),
            pltpu.SemaphoreType.DMA,
            pltpu.SemaphoreType.DMA,
        ],
    )(xs, src_f, dst_r, zeros_bd)


EPW = E // NW
PPW = 2 * E // NW
B2P = 128
C2P = 79
EPWP = C2P * B2P


def _spmm2_body(gsx_hbm, gsy_hbm, srcp_hbm, dstp_hbm, px_hbm, py_hbm,
                srcf_v, dst_v, gsx_v, gsy_v, rx_v, ry_v, rx2_v, ry2_v,
                zero_v, accx_s, accy_s, semx, semy, semx2, semy2):
    cid = lax.axis_index("c")
    sid = lax.axis_index("s")
    wid = cid * NS + sid

    _zero_fill(zero_v, RPT)
    pltpu.sync_copy(zero_v, accx_s.at[pl.ds(sid * RPT, RPT)])
    pltpu.sync_copy(zero_v, accy_s.at[pl.ds(sid * RPT, RPT)])
    pltpu.sync_copy(gsx_hbm, gsx_v)
    pltpu.sync_copy(gsy_hbm, gsy_v)
    pltpu.sync_copy(srcp_hbm.at[wid], srcf_v)
    pltpu.sync_copy(dstp_hbm.at[wid], dst_v)
    plsc.subcore_barrier()

    def build_fire(j, rx, ry, sx, sy):
        for m in range(B2P // L):
            s = pl.ds(m * L, L)
            si = srcf_v[pl.ds(j * B2P + m * L, L)]
            rx[s] = plsc.load_gather(gsx_v, [si])
            ry[s] = plsc.load_gather(gsy_v, [si])
        pltpu.async_copy(rx, accx_s.at[dst_v.at[j]], sx, add=True)
        pltpu.async_copy(ry, accy_s.at[dst_v.at[j]], sy, add=True)

    def chunk(j, carry):
        @pl.when(j % 2 == 0)
        def _():
            @pl.when(j >= 2)
            def _():
                pltpu.make_async_copy(rx_v, accx_s.at[dst_v.at[0]],
                                      semx).wait()
                pltpu.make_async_copy(ry_v, accy_s.at[dst_v.at[0]],
                                      semy).wait()
            build_fire(j, rx_v, ry_v, semx, semy)

        @pl.when(j % 2 == 1)
        def _():
            @pl.when(j >= 2)
            def _():
                pltpu.make_async_copy(rx2_v, accx_s.at[dst_v.at[0]],
                                      semx2).wait()
                pltpu.make_async_copy(ry2_v, accy_s.at[dst_v.at[0]],
                                      semy2).wait()
            build_fire(j, rx2_v, ry2_v, semx2, semy2)

        return carry

    lax.fori_loop(0, C2P, chunk, 0)
    pltpu.make_async_copy(rx_v, accx_s.at[dst_v.at[0]], semx).wait()
    pltpu.make_async_copy(ry_v, accy_s.at[dst_v.at[0]], semy).wait()
    pltpu.make_async_copy(rx2_v, accx_s.at[dst_v.at[0]], semx2).wait()
    pltpu.make_async_copy(ry2_v, accy_s.at[dst_v.at[0]], semy2).wait()
    plsc.subcore_barrier()

    rbase = sid * RPT
    pltpu.sync_copy(accx_s.at[pl.ds(rbase, RPT)], zero_v)
    pltpu.sync_copy(zero_v, px_hbm.at[pl.ds(cid * NP + rbase, RPT)])
    pltpu.sync_copy(accy_s.at[pl.ds(rbase, RPT)], zero_v)
    pltpu.sync_copy(zero_v, py_hbm.at[pl.ds(cid * NP + rbase, RPT)])


def _spmm2_partials(gsx, gsy, srcp, dstp):
    mesh = plsc.VectorSubcoreMesh(core_axis_name="c", subcore_axis_name="s")
    return pl.kernel(
        _spmm2_body,
        out_type=(
            jax.ShapeDtypeStruct((NC * NP,), jnp.float32),
            jax.ShapeDtypeStruct((NC * NP,), jnp.float32),
        ),
        mesh=mesh,
        compiler_params=pltpu.CompilerParams(needs_layout_passes=False),
        scratch_types=[
            pltpu.VMEM((EPWP,), jnp.int32),
            pltpu.VMEM((C2P, B2P), jnp.int32),
            pltpu.VMEM((NP,), jnp.float32),
            pltpu.VMEM((NP,), jnp.float32),
            pltpu.VMEM((B2P,), jnp.float32),
            pltpu.VMEM((B2P,), jnp.float32),
            pltpu.VMEM((B2P,), jnp.float32),
            pltpu.VMEM((B2P,), jnp.float32),
            pltpu.VMEM((RPT,), jnp.float32),
            pltpu.VMEM_SHARED((NP,), jnp.float32),
            pltpu.VMEM_SHARED((NP,), jnp.float32),
            pltpu.SemaphoreType.DMA,
            pltpu.SemaphoreType.DMA,
            pltpu.SemaphoreType.DMA,
            pltpu.SemaphoreType.DMA,
        ],
    )(gsx, gsy, srcp, dstp)


def _embdist_body(px_hbm, py_hbm, t2x_hbm, t2y_hbm, dinv_hbm,
                  src_hbm, dst_hbm, rn_hbm, cn_hbm,
                  embx_hbm, emby_hbm, ss_hbm,
                  ax_v, ay_v, bx_v, by_v, dv_v,
                  embx_s, emby_s, exf_v, eyf_v, paf_v, pbf_v, ssw_v):
    cid = lax.axis_index("c")
    sid = lax.axis_index("s")
    wid = cid * NS + sid

    rbase = sid * RPT
    pltpu.sync_copy(px_hbm.at[pl.ds(rbase, RPT)], ax_v)
    pltpu.sync_copy(px_hbm.at[pl.ds(NP + rbase, RPT)], bx_v)
    pltpu.sync_copy(py_hbm.at[pl.ds(rbase, RPT)], ay_v)
    pltpu.sync_copy(py_hbm.at[pl.ds(NP + rbase, RPT)], by_v)
    pltpu.sync_copy(dinv_hbm.at[pl.ds(rbase, RPT)], dv_v)

    def emb_rowx(k, carry):
        s = pl.ds(k * L, L)
        ax_v[s] = dv_v[s] * (ax_v[s] + bx_v[s])
        ay_v[s] = dv_v[s] * (ay_v[s] + by_v[s])
        return carry

    lax.fori_loop(0, RPT // L, emb_rowx, 0)
    pltpu.sync_copy(t2x_hbm.at[pl.ds(rbase, RPT)], bx_v)
    pltpu.sync_copy(t2y_hbm.at[pl.ds(rbase, RPT)], by_v)

    def emb_rowy(k, carry):
        s = pl.ds(k * L, L)
        ax_v[s] = ax_v[s] + bx_v[s]
        ay_v[s] = ay_v[s] + by_v[s]
        return carry

    lax.fori_loop(0, RPT // L, emb_rowy, 0)
    pltpu.sync_copy(ax_v, embx_s.at[pl.ds(rbase, RPT)])
    pltpu.sync_copy(ay_v, emby_s.at[pl.ds(rbase, RPT)])

    @pl.when(cid == 0)
    def _():
        pltpu.sync_copy(ax_v, embx_hbm.at[pl.ds(rbase, RPT)])
        pltpu.sync_copy(ay_v, emby_hbm.at[pl.ds(rbase, RPT)])

    plsc.subcore_barrier()

    pltpu.sync_copy(embx_s, exf_v)
    pltpu.sync_copy(emby_s, eyf_v)
    pbase = wid * PPW

    @pl.when(wid < NS)
    def _():
        pltpu.sync_copy(src_hbm.at[pl.ds(wid * PPW, PPW)], paf_v)
        pltpu.sync_copy(dst_hbm.at[pl.ds(wid * PPW, PPW)], pbf_v)

    @pl.when(wid >= NS)
    def _():
        pltpu.sync_copy(rn_hbm.at[pl.ds((wid - NS) * PPW, PPW)], paf_v)
        pltpu.sync_copy(cn_hbm.at[pl.ds((wid - NS) * PPW, PPW)], pbf_v)

    def pvec(v, carry):
        for r in range(5):
            s = pl.ds((v * 5 + r) * L, L)
            a = paf_v[s]
            b = pbf_v[s]
            dx = plsc.load_gather(exf_v, [a]) - plsc.load_gather(exf_v, [b])
            dy = plsc.load_gather(eyf_v, [a]) - plsc.load_gather(eyf_v, [b])
            ssw_v[s] = dx * dx + dy * dy
        return carry

    lax.fori_loop(0, PPW // L // 5, pvec, 0)
    pltpu.sync_copy(ssw_v, ss_hbm.at[pl.ds(pbase, PPW)])


def _embdist(px, py, t2x, t2y, dinv, src, dst, rn, cn):
    mesh = plsc.VectorSubcoreMesh(core_axis_name="c", subcore_axis_name="s")
    return pl.kernel(
        _embdist_body,
        out_type=(
            jax.ShapeDtypeStruct((NP,), jnp.float32),
            jax.ShapeDtypeStruct((NP,), jnp.float32),
            jax.ShapeDtypeStruct((2 * E,), jnp.float32),
        ),
        mesh=mesh,
        compiler_params=pltpu.CompilerParams(needs_layout_passes=False),
        scratch_types=[
            pltpu.VMEM((RPT,), jnp.float32),
            pltpu.VMEM((RPT,), jnp.float32),
            pltpu.VMEM((RPT,), jnp.float32),
            pltpu.VMEM((RPT,), jnp.float32),
            pltpu.VMEM((RPT,), jnp.float32),
            pltpu.VMEM_SHARED((NP,), jnp.float32),
            pltpu.VMEM_SHARED((NP,), jnp.float32),
            pltpu.VMEM((NP,), jnp.float32),
            pltpu.VMEM((NP,), jnp.float32),
            pltpu.VMEM((PPW,), jnp.int32),
            pltpu.VMEM((PPW,), jnp.int32),
            pltpu.VMEM((PPW,), jnp.float32),
        ],
    )(px, py, t2x, t2y, dinv, src, dst, rn, cn)


def _prep_body(degp_ref, x_ref, xs_ref, dinv_ref, dinv2_ref):
    deg = degp_ref[0, :] + degp_ref[1, :] + 1.0
    dinv = lax.rsqrt(deg)
    dinv_ref[...] = dinv
    dinv2_ref[...] = dinv * dinv
    xs_ref[pl.ds(0, N), :] = dinv[:N, None] * x_ref[...]
    xs_ref[pl.ds(N, NP - N), :] = jnp.zeros((NP - N, D), jnp.float32)


def _prep(deg_p, x):
    return pl.pallas_call(
        _prep_body,
        out_shape=(
            jax.ShapeDtypeStruct((NP, D), jnp.float32),
            jax.ShapeDtypeStruct((NP,), jnp.float32),
            jax.ShapeDtypeStruct((NP,), jnp.float32),
        ),
    )(deg_p.reshape(NC, NP), x)


_RB = 1024


def _mlp_body(p_ref, xs_ref, dinv_ref, dinv2_ref, w1_ref, b1_ref, w2_ref,
              b2_ref, gsx_ref, gsy_ref, t2x_ref, t2y_ref):
    dinv = dinv_ref[...]
    dinv2 = dinv2_ref[...]
    out1 = dinv[:, None] * (p_ref[0, :, :] + p_ref[1, :, :] + xs_ref[...])
    h = jnp.maximum(
        jnp.dot(out1, w1_ref[...], preferred_element_type=jnp.float32)
        + b1_ref[...], 0.0)
    g = jnp.dot(h, w2_ref[...], preferred_element_type=jnp.float32)
    gsx_ref[...] = dinv * g[:, 0]
    gsy_ref[...] = dinv * g[:, 1]
    t2x_ref[...] = dinv2 * g[:, 0] + b2_ref[0]
    t2y_ref[...] = dinv2 * g[:, 1] + b2_ref[1]


def _mlp(acc_p, xs, dinv, dinv2, W1, b1, W2, b2):
    grid = NP // _RB
    return pl.pallas_call(
        _mlp_body,
        grid=(grid,),
        in_specs=[
            pl.BlockSpec((NC, _RB, D), lambda i: (0, i, 0)),
            pl.BlockSpec((_RB, D), lambda i: (i, 0)),
            pl.BlockSpec((_RB,), lambda i: (i,)),
            pl.BlockSpec((_RB,), lambda i: (i,)),
            pl.BlockSpec((D, 256), lambda i: (0, 0)),
            pl.BlockSpec((256,), lambda i: (0,)),
            pl.BlockSpec((256, 2), lambda i: (0, 0)),
            pl.BlockSpec((2,), lambda i: (0,)),
        ],
        out_specs=[
            pl.BlockSpec((_RB,), lambda i: (i,)),
            pl.BlockSpec((_RB,), lambda i: (i,)),
            pl.BlockSpec((_RB,), lambda i: (i,)),
            pl.BlockSpec((_RB,), lambda i: (i,)),
        ],
        out_shape=[
            jax.ShapeDtypeStruct((NP,), jnp.float32),
            jax.ShapeDtypeStruct((NP,), jnp.float32),
            jax.ShapeDtypeStruct((NP,), jnp.float32),
            jax.ShapeDtypeStruct((NP,), jnp.float32),
        ],
    )(acc_p.reshape(NC, NP, D), xs, dinv, dinv2, W1, b1, W2, b2)


_QB = 128000


def _q_body(ss_ref, q_ref):
    ss = ss_ref[...]
    q_ref[...] = 1.0 / (1.0 + ALPHA * jnp.power(ss + 1e-12, BETA))


def _q_from_ss(ss):
    return pl.pallas_call(
        _q_body,
        grid=(2 * E // _QB,),
        in_specs=[pl.BlockSpec((_QB,), lambda i: (i,))],
        out_specs=pl.BlockSpec((_QB,), lambda i: (i,)),
        out_shape=jax.ShapeDtypeStruct((2 * E,), jnp.float32),
    )(ss)


def kernel(features, edge_index, row_neg, col_neg, W1, b1, W2, b2):
    src = edge_index[0]
    dst = edge_index[1]

    dst_r = dst.reshape(NW, C1, B1)
    src_f = src.reshape(NW, EPW)
    padrow = N + jnp.arange(EPWP - EPW, dtype=jnp.int32)
    padrows = jnp.broadcast_to(padrow, (NW, EPWP - EPW))
    srcp = jnp.concatenate([src.reshape(NW, EPW), padrows], axis=1)
    dstp = jnp.concatenate(
        [dst.reshape(NW, EPW), padrows], axis=1).reshape(NW, C2P, B2P)

    deg_p = _deg_partials(dst_r)
    xs, dinv, dinv2 = _prep(deg_p, features)

    acc_p = _spmm_partials(xs, src_f, dst_r, jnp.zeros((B1, D), jnp.float32))
    gsx, gsy, t2x, t2y = _mlp(acc_p, xs, dinv, dinv2, W1, b1, W2, b2)

    px, py = _spmm2_partials(gsx, gsy, srcp, dstp)
    embx, emby, ss = _embdist(px, py, t2x, t2y, dinv,
                              src, dst, row_neg, col_neg)
    emb = jnp.stack([embx[:N], emby[:N]], axis=1)

    q = _q_from_ss(ss)
    return (emb, q)


# speedup vs baseline: 3.2422x; 1.0006x over previous
import jax
import jax.numpy as jnp
from jax import lax
from jax.experimental import pallas as pl
from jax.experimental.pallas import tpu as pltpu
from jax.experimental.pallas import tpu_sc as plsc

ALPHA = 0.0813
BETA = 0.947

NC, NS, L = 2, 16, 16
NW = NC * NS

N = 10000
E = 320000
D = 128
NP = 10240
RPT = NP // NS
B1 = 80
C1 = (E // NW) // B1


def _zero_fill(ref, nwords):
    def body(i, c):
        ref[pl.ds(i * L, L)] = jnp.zeros((L,), jnp.float32)
        return c
    lax.fori_loop(0, nwords // L, body, 0)


def _deg_body(dst_hbm, deg_hbm, dst_v, ones_v, zero_v, acc_s, bounce_v, sem):
    cid = lax.axis_index("c")
    sid = lax.axis_index("s")
    wid = cid * NS + sid

    _zero_fill(zero_v, RPT)
    pltpu.sync_copy(zero_v, acc_s.at[pl.ds(sid * RPT, RPT)])
    plsc.subcore_barrier()

    pltpu.sync_copy(dst_hbm.at[wid], dst_v)
    for i in range(0, B1, L):
        ones_v[pl.ds(i, L)] = jnp.ones((L,), jnp.float32)

    def chunk(j, carry):
        pltpu.async_copy(ones_v, acc_s.at[dst_v.at[j]], sem, add=True)

        @pl.when(j >= 16)
        def _():
            pltpu.make_async_copy(ones_v, acc_s.at[dst_v.at[0]], sem).wait()
        return carry

    lax.fori_loop(0, C1, chunk, 0)

    def drain(j, carry):
        pltpu.make_async_copy(ones_v, acc_s.at[dst_v.at[0]], sem).wait()
        return carry

    lax.fori_loop(0, 16, drain, 0)
    plsc.subcore_barrier()

    pltpu.sync_copy(acc_s.at[pl.ds(sid * RPT, RPT)], bounce_v)
    pltpu.sync_copy(bounce_v, deg_hbm.at[pl.ds(cid * NP + sid * RPT, RPT)])


def _deg_partials(dst_r):
    mesh = plsc.VectorSubcoreMesh(core_axis_name="c", subcore_axis_name="s")
    return pl.kernel(
        _deg_body,
        out_type=jax.ShapeDtypeStruct((NC * NP,), jnp.float32),
        mesh=mesh,
        compiler_params=pltpu.CompilerParams(needs_layout_passes=False),
        scratch_types=[
            pltpu.VMEM((C1, B1), jnp.int32),
            pltpu.VMEM((B1,), jnp.float32),
            pltpu.VMEM((RPT,), jnp.float32),
            pltpu.VMEM_SHARED((NP,), jnp.float32),
            pltpu.VMEM((RPT,), jnp.float32),
            pltpu.SemaphoreType.DMA,
        ],
    )(dst_r)


def _spmm_body(xs_hbm, srcf_hbm, dst_hbm, zeros_hbm, out_hbm,
               srcf_v, dst_v, r2, rb, acc_s, sem, semb):
    cid = lax.axis_index("c")
    sid = lax.axis_index("s")
    wid = cid * NS + sid

    pltpu.sync_copy(zeros_hbm, r2)
    for t in range(RPT // B1):
        pltpu.sync_copy(r2, acc_s.at[pl.ds(sid * RPT + t * B1, B1)])
    plsc.subcore_barrier()

    pltpu.sync_copy(srcf_hbm.at[wid], srcf_v)
    pltpu.sync_copy(dst_hbm.at[wid], dst_v)

    pltpu.async_copy(xs_hbm.at[srcf_v.at[pl.ds(0, B1)]], r2, sem)

    def chunk(j, carry):
        sbase = j * B1

        @pl.when(j % 2 == 0)
        def _():
            @pl.when(j + 1 < C1)
            def _():
                pltpu.async_copy(
                    xs_hbm.at[srcf_v.at[pl.ds(sbase + B1, B1)]], rb, semb)
            pltpu.make_async_copy(xs_hbm.at[pl.ds(0, B1)], r2, sem).wait()
            pltpu.sync_copy(r2, acc_s.at[dst_v.at[j]], add=True)

        @pl.when(j % 2 == 1)
        def _():
            @pl.when(j + 1 < C1)
            def _():
                pltpu.async_copy(
                    xs_hbm.at[srcf_v.at[pl.ds(sbase + B1, B1)]], r2, sem)
            pltpu.make_async_copy(xs_hbm.at[pl.ds(0, B1)], rb, semb).wait()
            pltpu.sync_copy(rb, acc_s.at[dst_v.at[j]], add=True)

        return carry

    lax.fori_loop(0, C1, chunk, 0)
    plsc.subcore_barrier()

    for t in range(RPT // B1):
        base = sid * RPT + t * B1
        pltpu.sync_copy(acc_s.at[pl.ds(base, B1)], r2)
        pltpu.sync_copy(r2, out_hbm.at[pl.ds(cid * NP + base, B1)])


def _spmm_partials(xs, src_f, dst_r, zeros_bd):
    mesh = plsc.VectorSubcoreMesh(core_axis_name="c", subcore_axis_name="s")
    return pl.kernel(
        _spmm_body,
        out_type=jax.ShapeDtypeStruct((NC * NP, D), jnp.float32),
        mesh=mesh,
        compiler_params=pltpu.CompilerParams(needs_layout_passes=False),
        scratch_types=[
            pltpu.VMEM((E // NW,), jnp.int32),
            pltpu.VMEM((C1, B1), jnp.int32),
            pltpu.VMEM((B1, D), jnp.float32),
            pltpu.VMEM((B1, D), jnp.float32),
            pltpu.VMEM_SHARED((NP, D), jnp.float32),
            pltpu.SemaphoreType.DMA,
            pltpu.SemaphoreType.DMA,
        ],
    )(xs, src_f, dst_r, zeros_bd)


EPW = E // NW
PPW = 2 * E // NW
B2P = 128
C2P = 79
EPWP = C2P * B2P


def _spmm2_body(gsx_hbm, gsy_hbm, srcp_hbm, dstp_hbm, px_hbm, py_hbm,
                srcf_v, dst_v, gsx_v, gsy_v, rx_v, ry_v, rx2_v, ry2_v,
                zero_v, accx_s, accy_s, semx, semy, semx2, semy2):
    cid = lax.axis_index("c")
    sid = lax.axis_index("s")
    wid = cid * NS + sid

    _zero_fill(zero_v, RPT)
    pltpu.sync_copy(zero_v, accx_s.at[pl.ds(sid * RPT, RPT)])
    pltpu.sync_copy(zero_v, accy_s.at[pl.ds(sid * RPT, RPT)])
    pltpu.sync_copy(gsx_hbm, gsx_v)
    pltpu.sync_copy(gsy_hbm, gsy_v)
    pltpu.sync_copy(srcp_hbm.at[wid], srcf_v)
    pltpu.sync_copy(dstp_hbm.at[wid], dst_v)
    plsc.subcore_barrier()

    def build_fire(j, rx, ry, sx, sy):
        for m in range(B2P // L):
            s = pl.ds(m * L, L)
            si = srcf_v[pl.ds(j * B2P + m * L, L)]
            rx[s] = plsc.load_gather(gsx_v, [si])
            ry[s] = plsc.load_gather(gsy_v, [si])
        pltpu.async_copy(rx, accx_s.at[dst_v.at[j]], sx, add=True)
        pltpu.async_copy(ry, accy_s.at[dst_v.at[j]], sy, add=True)

    def chunk(j, carry):
        @pl.when(j % 2 == 0)
        def _():
            @pl.when(j >= 2)
            def _():
                pltpu.make_async_copy(rx_v, accx_s.at[dst_v.at[0]],
                                      semx).wait()
                pltpu.make_async_copy(ry_v, accy_s.at[dst_v.at[0]],
                                      semy).wait()
            build_fire(j, rx_v, ry_v, semx, semy)

        @pl.when(j % 2 == 1)
        def _():
            @pl.when(j >= 2)
            def _():
                pltpu.make_async_copy(rx2_v, accx_s.at[dst_v.at[0]],
                                      semx2).wait()
                pltpu.make_async_copy(ry2_v, accy_s.at[dst_v.at[0]],
                                      semy2).wait()
            build_fire(j, rx2_v, ry2_v, semx2, semy2)

        return carry

    lax.fori_loop(0, C2P, chunk, 0)
    pltpu.make_async_copy(rx_v, accx_s.at[dst_v.at[0]], semx).wait()
    pltpu.make_async_copy(ry_v, accy_s.at[dst_v.at[0]], semy).wait()
    pltpu.make_async_copy(rx2_v, accx_s.at[dst_v.at[0]], semx2).wait()
    pltpu.make_async_copy(ry2_v, accy_s.at[dst_v.at[0]], semy2).wait()
    plsc.subcore_barrier()

    rbase = sid * RPT
    pltpu.sync_copy(accx_s.at[pl.ds(rbase, RPT)], zero_v)
    pltpu.sync_copy(zero_v, px_hbm.at[pl.ds(cid * NP + rbase, RPT)])
    pltpu.sync_copy(accy_s.at[pl.ds(rbase, RPT)], zero_v)
    pltpu.sync_copy(zero_v, py_hbm.at[pl.ds(cid * NP + rbase, RPT)])


def _spmm2_partials(gsx, gsy, srcp, dstp):
    mesh = plsc.VectorSubcoreMesh(core_axis_name="c", subcore_axis_name="s")
    return pl.kernel(
        _spmm2_body,
        out_type=(
            jax.ShapeDtypeStruct((NC * NP,), jnp.float32),
            jax.ShapeDtypeStruct((NC * NP,), jnp.float32),
        ),
        mesh=mesh,
        compiler_params=pltpu.CompilerParams(needs_layout_passes=False),
        scratch_types=[
            pltpu.VMEM((EPWP,), jnp.int32),
            pltpu.VMEM((C2P, B2P), jnp.int32),
            pltpu.VMEM((NP,), jnp.float32),
            pltpu.VMEM((NP,), jnp.float32),
            pltpu.VMEM((B2P,), jnp.float32),
            pltpu.VMEM((B2P,), jnp.float32),
            pltpu.VMEM((B2P,), jnp.float32),
            pltpu.VMEM((B2P,), jnp.float32),
            pltpu.VMEM((RPT,), jnp.float32),
            pltpu.VMEM_SHARED((NP,), jnp.float32),
            pltpu.VMEM_SHARED((NP,), jnp.float32),
            pltpu.SemaphoreType.DMA,
            pltpu.SemaphoreType.DMA,
            pltpu.SemaphoreType.DMA,
            pltpu.SemaphoreType.DMA,
        ],
    )(gsx, gsy, srcp, dstp)


def _embdist_body(px_hbm, py_hbm, t2x_hbm, t2y_hbm, dinv_hbm,
                  src_hbm, dst_hbm, rn_hbm, cn_hbm,
                  embx_hbm, emby_hbm, ss_hbm,
                  ax_v, ay_v, bx_v, by_v, dv_v,
                  embx_s, emby_s, exf_v, eyf_v, paf_v, pbf_v, ssw_v):
    cid = lax.axis_index("c")
    sid = lax.axis_index("s")
    wid = cid * NS + sid

    rbase = sid * RPT
    pltpu.sync_copy(px_hbm.at[pl.ds(rbase, RPT)], ax_v)
    pltpu.sync_copy(px_hbm.at[pl.ds(NP + rbase, RPT)], bx_v)
    pltpu.sync_copy(py_hbm.at[pl.ds(rbase, RPT)], ay_v)
    pltpu.sync_copy(py_hbm.at[pl.ds(NP + rbase, RPT)], by_v)
    pltpu.sync_copy(dinv_hbm.at[pl.ds(rbase, RPT)], dv_v)

    def emb_rowx(k, carry):
        s = pl.ds(k * L, L)
        ax_v[s] = dv_v[s] * (ax_v[s] + bx_v[s])
        ay_v[s] = dv_v[s] * (ay_v[s] + by_v[s])
        return carry

    lax.fori_loop(0, RPT // L, emb_rowx, 0)
    pltpu.sync_copy(t2x_hbm.at[pl.ds(rbase, RPT)], bx_v)
    pltpu.sync_copy(t2y_hbm.at[pl.ds(rbase, RPT)], by_v)

    def emb_rowy(k, carry):
        s = pl.ds(k * L, L)
        ax_v[s] = ax_v[s] + bx_v[s]
        ay_v[s] = ay_v[s] + by_v[s]
        return carry

    lax.fori_loop(0, RPT // L, emb_rowy, 0)
    pltpu.sync_copy(ax_v, embx_s.at[pl.ds(rbase, RPT)])
    pltpu.sync_copy(ay_v, emby_s.at[pl.ds(rbase, RPT)])

    @pl.when(cid == 0)
    def _():
        pltpu.sync_copy(ax_v, embx_hbm.at[pl.ds(rbase, RPT)])
        pltpu.sync_copy(ay_v, emby_hbm.at[pl.ds(rbase, RPT)])

    plsc.subcore_barrier()

    pltpu.sync_copy(embx_s, exf_v)
    pltpu.sync_copy(emby_s, eyf_v)
    pbase = wid * PPW

    @pl.when(wid < NS)
    def _():
        pltpu.sync_copy(src_hbm.at[pl.ds(wid * PPW, PPW)], paf_v)
        pltpu.sync_copy(dst_hbm.at[pl.ds(wid * PPW, PPW)], pbf_v)

    @pl.when(wid >= NS)
    def _():
        pltpu.sync_copy(rn_hbm.at[pl.ds((wid - NS) * PPW, PPW)], paf_v)
        pltpu.sync_copy(cn_hbm.at[pl.ds((wid - NS) * PPW, PPW)], pbf_v)

    def pvec(v, carry):
        for r in range(5):
            s = pl.ds((v * 5 + r) * L, L)
            a = paf_v[s]
            b = pbf_v[s]
            dx = plsc.load_gather(exf_v, [a]) - plsc.load_gather(exf_v, [b])
            dy = plsc.load_gather(eyf_v, [a]) - plsc.load_gather(eyf_v, [b])
            ssw_v[s] = dx * dx + dy * dy
        return carry

    lax.fori_loop(0, PPW // L // 5, pvec, 0)
    pltpu.sync_copy(ssw_v, ss_hbm.at[pl.ds(pbase, PPW)])


def _embdist(px, py, t2x, t2y, dinv, src, dst, rn, cn):
    mesh = plsc.VectorSubcoreMesh(core_axis_name="c", subcore_axis_name="s")
    return pl.kernel(
        _embdist_body,
        out_type=(
            jax.ShapeDtypeStruct((NP,), jnp.float32),
            jax.ShapeDtypeStruct((NP,), jnp.float32),
            jax.ShapeDtypeStruct((2 * E,), jnp.float32),
        ),
        mesh=mesh,
        compiler_params=pltpu.CompilerParams(needs_layout_passes=False),
        scratch_types=[
            pltpu.VMEM((RPT,), jnp.float32),
            pltpu.VMEM((RPT,), jnp.float32),
            pltpu.VMEM((RPT,), jnp.float32),
            pltpu.VMEM((RPT,), jnp.float32),
            pltpu.VMEM((RPT,), jnp.float32),
            pltpu.VMEM_SHARED((NP,), jnp.float32),
            pltpu.VMEM_SHARED((NP,), jnp.float32),
            pltpu.VMEM((NP,), jnp.float32),
            pltpu.VMEM((NP,), jnp.float32),
            pltpu.VMEM((PPW,), jnp.int32),
            pltpu.VMEM((PPW,), jnp.int32),
            pltpu.VMEM((PPW,), jnp.float32),
        ],
    )(px, py, t2x, t2y, dinv, src, dst, rn, cn)


def _prep_body(degp_ref, x_ref, xs_ref, dinv_ref, dinv2_ref):
    deg = degp_ref[0, :] + degp_ref[1, :] + 1.0
    dinv = lax.rsqrt(deg)
    dinv_ref[...] = dinv
    dinv2_ref[...] = dinv * dinv
    xs_ref[pl.ds(0, N), :] = dinv[:N, None] * x_ref[...]
    xs_ref[pl.ds(N, NP - N), :] = jnp.zeros((NP - N, D), jnp.float32)


def _prep(deg_p, x):
    return pl.pallas_call(
        _prep_body,
        out_shape=(
            jax.ShapeDtypeStruct((NP, D), jnp.float32),
            jax.ShapeDtypeStruct((NP,), jnp.float32),
            jax.ShapeDtypeStruct((NP,), jnp.float32),
        ),
    )(deg_p.reshape(NC, NP), x)


_RB = 1024


def _mlp_body(p_ref, xs_ref, dinv_ref, dinv2_ref, w1_ref, b1_ref, w2_ref,
              b2_ref, gsx_ref, gsy_ref, t2x_ref, t2y_ref):
    dinv = dinv_ref[...]
    dinv2 = dinv2_ref[...]
    out1 = dinv[:, None] * (p_ref[0, :, :] + p_ref[1, :, :] + xs_ref[...])
    h = jnp.maximum(
        jnp.dot(out1, w1_ref[...], preferred_element_type=jnp.float32)
        + b1_ref[...], 0.0)
    g = jnp.dot(h, w2_ref[...], preferred_element_type=jnp.float32)
    gsx_ref[...] = dinv * g[:, 0]
    gsy_ref[...] = dinv * g[:, 1]
    t2x_ref[...] = dinv2 * g[:, 0] + b2_ref[0]
    t2y_ref[...] = dinv2 * g[:, 1] + b2_ref[1]


def _mlp(acc_p, xs, dinv, dinv2, W1, b1, W2, b2):
    grid = NP // _RB
    return pl.pallas_call(
        _mlp_body,
        grid=(grid,),
        in_specs=[
            pl.BlockSpec((NC, _RB, D), lambda i: (0, i, 0)),
            pl.BlockSpec((_RB, D), lambda i: (i, 0)),
            pl.BlockSpec((_RB,), lambda i: (i,)),
            pl.BlockSpec((_RB,), lambda i: (i,)),
            pl.BlockSpec((D, 256), lambda i: (0, 0)),
            pl.BlockSpec((256,), lambda i: (0,)),
            pl.BlockSpec((256, 2), lambda i: (0, 0)),
            pl.BlockSpec((2,), lambda i: (0,)),
        ],
        out_specs=[
            pl.BlockSpec((_RB,), lambda i: (i,)),
            pl.BlockSpec((_RB,), lambda i: (i,)),
            pl.BlockSpec((_RB,), lambda i: (i,)),
            pl.BlockSpec((_RB,), lambda i: (i,)),
        ],
        out_shape=[
            jax.ShapeDtypeStruct((NP,), jnp.float32),
            jax.ShapeDtypeStruct((NP,), jnp.float32),
            jax.ShapeDtypeStruct((NP,), jnp.float32),
            jax.ShapeDtypeStruct((NP,), jnp.float32),
        ],
    )(acc_p.reshape(NC, NP, D), xs, dinv, dinv2, W1, b1, W2, b2)


_QB = 128000


def _q_body(ss_ref, q_ref):
    ss = ss_ref[...]
    q_ref[...] = 1.0 / (1.0 + ALPHA * jnp.power(ss + 1e-12, BETA))


def _q_from_ss(ss):
    return pl.pallas_call(
        _q_body,
        grid=(2 * E // _QB,),
        in_specs=[pl.BlockSpec((_QB,), lambda i: (i,))],
        out_specs=pl.BlockSpec((_QB,), lambda i: (i,)),
        out_shape=jax.ShapeDtypeStruct((2 * E,), jnp.float32),
    )(ss)


def kernel(features, edge_index, row_neg, col_neg, W1, b1, W2, b2):
    src = edge_index[0]
    dst = edge_index[1]

    dst_r = dst.reshape(NW, C1, B1)
    src_f = src.reshape(NW, EPW)
    padrow = N + jnp.arange(EPWP - EPW, dtype=jnp.int32)
    padrows = jnp.broadcast_to(padrow, (NW, EPWP - EPW))
    srcp = jnp.concatenate([src.reshape(NW, EPW), padrows], axis=1)
    dstp = jnp.concatenate(
        [dst.reshape(NW, EPW), padrows], axis=1).reshape(NW, C2P, B2P)

    deg_p = _deg_partials(dst_r)
    xs, dinv, dinv2 = _prep(deg_p, features)

    acc_p = _spmm_partials(xs, src_f, dst_r, jnp.zeros((B1, D), jnp.float32))
    gsx, gsy, t2x, t2y = _mlp(acc_p, xs, dinv, dinv2, W1, b1, W2, b2)

    px, py = _spmm2_partials(gsx, gsy, srcp, dstp)
    embx, emby, ss = _embdist(px, py, t2x, t2y, dinv,
                              src, dst, row_neg, col_neg)
    emb = jnp.stack([embx[:N], emby[:N]], axis=1)

    q = _q_from_ss(ss)
    return (emb, q)
